# baseline (speedup 1.0000x reference)
import functools

import jax
import jax.numpy as jnp
import numpy as np
from jax import lax
from jax.experimental import pallas as pl
from jax.experimental.pallas import tpu as pltpu
from jax.experimental.pallas import tpu_sc as plsc

_EPS = 1e-5


def _logits_kernel(x_ref, rw_ref, rb_ref, lt_ref):
    logits = jnp.dot(x_ref[...], rw_ref[...],
                     preferred_element_type=jnp.float32)
    logits = logits + rb_ref[...]
    lt_ref[...] = logits.T


def _sc_routing_body(lt_hbm, gates_hbm, buf, obuf, *, E, TPW, k_top, nc):
    wid = lax.axis_index("s") * nc + lax.axis_index("c")
    base = wid * TPW
    pltpu.sync_copy(lt_hbm.at[:, pl.ds(base, TPW)], buf)

    @pl.loop(0, TPW // 16)
    def _group(g):
            sl = pl.ds(g * 16, 16)
            l = [buf[j, sl] for j in range(E)]
            orig = list(l)
            sel = [None] * E
            mx = None
            for k in range(k_top):
                m = l[0]
                for j in range(1, E):
                    m = jnp.maximum(m, l[j])
                if k == 0:
                    mx = m
                idxm = jnp.full((16,), 1.0e9, jnp.float32)
                for j in range(E):
                    idxm = jnp.minimum(
                        idxm, jnp.where(l[j] == m, float(j), 1.0e9))
                for j in range(E):
                    pick = (l[j] == m) & (idxm == float(j))
                    sel[j] = pick if sel[j] is None else (sel[j] | pick)
                    l[j] = jnp.where(pick, -1.0e30, l[j])
            ssum = jnp.zeros((16,), jnp.float32)
            es = []
            for j in range(E):
                e = jnp.where(sel[j], jnp.exp(orig[j] - mx), 0.0)
                es.append(e)
                ssum = ssum + e
            inv = 1.0 / ssum
            for j in range(E):
                obuf[j, sl] = es[j] * inv
    pltpu.sync_copy(obuf, gates_hbm.at[:, pl.ds(base, TPW)])


def _sc_routing(logits_t, k_top):
    E, T = logits_t.shape
    info = plsc.get_sparse_core_info()
    nc, ns = info.num_cores, info.num_subcores
    nw = nc * ns
    TPW = T // nw
    mesh = plsc.VectorSubcoreMesh(core_axis_name="c", subcore_axis_name="s")
    fn = functools.partial(
        pl.kernel,
        mesh=mesh,
        out_type=jax.ShapeDtypeStruct((E, T), jnp.float32),
        scratch_types=[
            pltpu.VMEM((E, TPW), jnp.float32),
            pltpu.VMEM((E, TPW), jnp.float32),
        ],
    )(functools.partial(_sc_routing_body, E=E, TPW=TPW, k_top=k_top, nc=nc))
    return fn(logits_t)


def _col_stats(h):
    n = h.shape[0]
    m = jnp.mean(h, axis=0, keepdims=True)
    v = jnp.mean(jnp.square(h - m), axis=0, keepdims=True)
    return m, v


def _layer1_kernel(x_ref, w1_ref, a_ref, h_scr):
    j = pl.program_id(0)
    h = jnp.dot(x_ref[...], w1_ref[...], preferred_element_type=jnp.float32)
    h_scr[j % 2] = h
    hp = h_scr[(j + 1) % 2]
    m, v = _col_stats(hp)
    a = jnp.maximum((hp - m) * jax.lax.rsqrt(v + _EPS), 0.0)
    a_ref[...] = a.astype(a_ref.dtype)


def _layer2_kernel(cc_ref, first_ref, last_ref, eid_ref,
                   a_ref, w2_ref, gates_ref, out_ref, h2_ref, *, nd):
    s = pl.program_id(0)
    D = w2_ref.shape[1]
    dw = D // nd

    @pl.when(first_ref[s] == 1)
    def _():
        h2_ref[...] = jnp.zeros_like(h2_ref)

    @pl.when(s == 0)
    def _():
        out_ref[...] = jnp.zeros_like(out_ref)

    a_blk = a_ref[...]
    for dh in range(nd):
        h2_ref[:, dh * dw:(dh + 1) * dw] += jnp.dot(
            a_blk, w2_ref[:, dh * dw:(dh + 1) * dw],
            preferred_element_type=jnp.float32)

    @pl.when(last_ref[s] == 1)
    def _():
        g = gates_ref[0]
        for dh in range(nd):
            h2 = h2_ref[:, dh * dw:(dh + 1) * dw]
            m, v = _col_stats(h2)
            bn = (h2 - m) * jax.lax.rsqrt(v + _EPS)
            out_ref[:, dh * dw:(dh + 1) * dw] += g * bn


def _moe_forward(x, router_w, router_b, w1_list, w2_list, *,
                 k_top, pad, token_block, interpret=False):
    T, D = x.shape
    E = len(w1_list)
    sizes = [int(w.shape[1]) for w in w1_list]
    psizes = [-(-s // pad) * pad for s in sizes]
    S = int(sum(psizes))
    nchunks = [ps // pad for ps in psizes]
    NC = int(sum(nchunks))

    logits_t = pl.pallas_call(
        _logits_kernel,
        grid=(T // token_block,),
        in_specs=[
            pl.BlockSpec((token_block, D), lambda i: (i, 0)),
            pl.BlockSpec((D, E), lambda i: (0, 0)),
            pl.BlockSpec((1, E), lambda i: (0, 0)),
        ],
        out_specs=pl.BlockSpec((E, token_block), lambda i: (0, i)),
        out_shape=jax.ShapeDtypeStruct((E, T), jnp.float32),
        interpret=interpret,
    )(x, router_w, router_b.reshape(1, E))
    gates_t = _sc_routing(logits_t, k_top)

    W1p = jnp.concatenate(
        [jnp.pad(w, ((0, 0), (0, ps - s)))
         for w, s, ps in zip(w1_list, sizes, psizes)],
        axis=1).astype(jnp.bfloat16)
    W2p = jnp.concatenate(
        [jnp.pad(w, ((0, ps - s), (0, 0)))
         for w, s, ps in zip(w2_list, sizes, psizes)],
        axis=0).astype(jnp.bfloat16)
    x_bf = x.astype(jnp.bfloat16)

    a = pl.pallas_call(
        _layer1_kernel,
        grid=(NC + 1,),
        in_specs=[
            pl.BlockSpec((T, D), lambda j: (0, 0)),
            pl.BlockSpec((D, pad), lambda j: (0, jnp.minimum(j, NC - 1))),
        ],
        out_specs=pl.BlockSpec(
            (T, pad), lambda j: (0, jnp.maximum(j - 1, 0))),
        out_shape=jax.ShapeDtypeStruct((T, S), jnp.bfloat16),
        scratch_shapes=[pltpu.VMEM((2, T, pad), jnp.float32)],
        interpret=interpret,
    )(x_bf, W1p)

    cc, eid, first, last = [], [], [], []
    for e in range(E):
        base = sum(nchunks[:e])
        for j in range(nchunks[e]):
            cc.append(base + j)
            eid.append(e)
            first.append(1 if j == 0 else 0)
            last.append(1 if j == nchunks[e] - 1 else 0)
    cc = jnp.asarray(np.asarray(cc, np.int32))
    eid = jnp.asarray(np.asarray(eid, np.int32))
    first = jnp.asarray(np.asarray(first, np.int32))
    last = jnp.asarray(np.asarray(last, np.int32))

    grid_spec = pltpu.PrefetchScalarGridSpec(
        num_scalar_prefetch=4,
        grid=(NC,),
        in_specs=[
            pl.BlockSpec((T, pad), lambda s, cc, fr, la, ei: (0, cc[s])),
            pl.BlockSpec((pad, D), lambda s, cc, fr, la, ei: (cc[s], 0)),
            pl.BlockSpec((1, T, 1), lambda s, cc, fr, la, ei: (ei[s], 0, 0)),
        ],
        out_specs=pl.BlockSpec((T, D), lambda s, cc, fr, la, ei: (0, 0)),
        scratch_shapes=[pltpu.VMEM((T, D), jnp.float32)],
    )
    out = pl.pallas_call(
        functools.partial(_layer2_kernel, nd=max(1, D // 512)),
        grid_spec=grid_spec,
        out_shape=jax.ShapeDtypeStruct((T, D), jnp.float32),
        compiler_params=pltpu.CompilerParams(
            vmem_limit_bytes=63 * 1024 * 1024),
        interpret=interpret,
    )(cc, first, last, eid, a, W2p, gates_t.reshape(E, T, 1))
    return out + x


def kernel(x, router_w, router_b, *expert_params):
    w1_list = expert_params[0::4]
    w2_list = expert_params[2::4]
    return _moe_forward(x, router_w, router_b, list(w1_list), list(w2_list),
                        k_top=8, pad=256, token_block=512)


# speedup vs baseline: 1.0754x; 1.0754x over previous
import functools

import jax
import jax.numpy as jnp
import numpy as np
from jax import lax
from jax.experimental import pallas as pl
from jax.experimental.pallas import tpu as pltpu
from jax.experimental.pallas import tpu_sc as plsc

_EPS = 1e-5


def _logits_kernel(x_ref, rw_ref, rb_ref, lt_ref):
    logits = jnp.dot(x_ref[...], rw_ref[...],
                     preferred_element_type=jnp.float32)
    logits = logits + rb_ref[...]
    lt_ref[...] = logits.T


def _sc_routing_body(lt_hbm, gates_hbm, buf, obuf, *, E, TPW, k_top, nc):
    wid = lax.axis_index("s") * nc + lax.axis_index("c")
    base = wid * TPW
    pltpu.sync_copy(lt_hbm.at[:, pl.ds(base, TPW)], buf)

    @pl.loop(0, TPW // 16)
    def _group(g):
            sl = pl.ds(g * 16, 16)
            l = [buf[j, sl] for j in range(E)]
            orig = list(l)
            sel = [None] * E
            mx = None
            for k in range(k_top):
                m = l[0]
                for j in range(1, E):
                    m = jnp.maximum(m, l[j])
                if k == 0:
                    mx = m
                idxm = jnp.full((16,), 1.0e9, jnp.float32)
                for j in range(E):
                    idxm = jnp.minimum(
                        idxm, jnp.where(l[j] == m, float(j), 1.0e9))
                for j in range(E):
                    pick = (l[j] == m) & (idxm == float(j))
                    sel[j] = pick if sel[j] is None else (sel[j] | pick)
                    l[j] = jnp.where(pick, -1.0e30, l[j])
            ssum = jnp.zeros((16,), jnp.float32)
            es = []
            for j in range(E):
                e = jnp.where(sel[j], jnp.exp(orig[j] - mx), 0.0)
                es.append(e)
                ssum = ssum + e
            inv = 1.0 / ssum
            for j in range(E):
                obuf[j, sl] = es[j] * inv
    pltpu.sync_copy(obuf, gates_hbm.at[:, pl.ds(base, TPW)])


def _sc_routing(logits_t, k_top):
    E, T = logits_t.shape
    info = plsc.get_sparse_core_info()
    nc, ns = info.num_cores, info.num_subcores
    nw = nc * ns
    TPW = T // nw
    mesh = plsc.VectorSubcoreMesh(core_axis_name="c", subcore_axis_name="s")
    fn = functools.partial(
        pl.kernel,
        mesh=mesh,
        out_type=jax.ShapeDtypeStruct((E, T), jnp.float32),
        scratch_types=[
            pltpu.VMEM((E, TPW), jnp.float32),
            pltpu.VMEM((E, TPW), jnp.float32),
        ],
    )(functools.partial(_sc_routing_body, E=E, TPW=TPW, k_top=k_top, nc=nc))
    return fn(logits_t)


def _col_stats(h):
    n = h.shape[0]
    m = jnp.mean(h, axis=0, keepdims=True)
    v = jnp.mean(jnp.square(h - m), axis=0, keepdims=True)
    return m, v


def _layer1_kernel(x_ref, w1_ref, a_ref):
    h = jnp.dot(x_ref[...], w1_ref[...], preferred_element_type=jnp.float32)
    m, v = _col_stats(h)
    a = jnp.maximum((h - m) * jax.lax.rsqrt(v + _EPS), 0.0)
    a_ref[...] = a.astype(a_ref.dtype)


def _layer2_kernel(cc_ref, first_ref, last_ref, eid_ref,
                   a_ref, w2_ref, gates_ref, out_ref, h2_ref, *, nd):
    s = pl.program_id(0)
    D = w2_ref.shape[1]
    dw = D // nd

    @pl.when(first_ref[s] == 1)
    def _():
        h2_ref[...] = jnp.zeros_like(h2_ref)

    @pl.when(s == 0)
    def _():
        out_ref[...] = jnp.zeros_like(out_ref)

    a_blk = a_ref[...]
    for dh in range(nd):
        h2_ref[:, dh * dw:(dh + 1) * dw] += jnp.dot(
            a_blk, w2_ref[:, dh * dw:(dh + 1) * dw],
            preferred_element_type=jnp.float32)

    @pl.when(last_ref[s] == 1)
    def _():
        g = gates_ref[0]
        for dh in range(nd):
            h2 = h2_ref[:, dh * dw:(dh + 1) * dw]
            m, v = _col_stats(h2)
            bn = (h2 - m) * jax.lax.rsqrt(v + _EPS)
            out_ref[:, dh * dw:(dh + 1) * dw] += g * bn


def _moe_forward(x, router_w, router_b, w1_list, w2_list, *,
                 k_top, pad, token_block, interpret=False):
    T, D = x.shape
    E = len(w1_list)
    sizes = [int(w.shape[1]) for w in w1_list]
    psizes = [-(-s // pad) * pad for s in sizes]
    S = int(sum(psizes))
    nchunks = [ps // pad for ps in psizes]
    NC = int(sum(nchunks))

    logits_t = pl.pallas_call(
        _logits_kernel,
        grid=(T // token_block,),
        in_specs=[
            pl.BlockSpec((token_block, D), lambda i: (i, 0)),
            pl.BlockSpec((D, E), lambda i: (0, 0)),
            pl.BlockSpec((1, E), lambda i: (0, 0)),
        ],
        out_specs=pl.BlockSpec((E, token_block), lambda i: (0, i)),
        out_shape=jax.ShapeDtypeStruct((E, T), jnp.float32),
        interpret=interpret,
    )(x, router_w, router_b.reshape(1, E))

    W1p = jnp.concatenate(
        [jnp.pad(w, ((0, 0), (0, ps - s)))
         for w, s, ps in zip(w1_list, sizes, psizes)],
        axis=1).astype(jnp.bfloat16)
    W2p = jnp.concatenate(
        [jnp.pad(w, ((0, ps - s), (0, 0)))
         for w, s, ps in zip(w2_list, sizes, psizes)],
        axis=0).astype(jnp.bfloat16)
    x_bf = x.astype(jnp.bfloat16)

    a = pl.pallas_call(
        _layer1_kernel,
        grid=(NC,),
        in_specs=[
            pl.BlockSpec((T, D), lambda j: (0, 0)),
            pl.BlockSpec((D, pad), lambda j: (0, j)),
        ],
        out_specs=pl.BlockSpec((T, pad), lambda j: (0, j)),
        out_shape=jax.ShapeDtypeStruct((T, S), jnp.bfloat16),
        interpret=interpret,
    )(x_bf, W1p)

    gates_t = _sc_routing(logits_t, k_top)

    cc, eid, first, last = [], [], [], []
    for e in range(E):
        base = sum(nchunks[:e])
        for j in range(nchunks[e]):
            cc.append(base + j)
            eid.append(e)
            first.append(1 if j == 0 else 0)
            last.append(1 if j == nchunks[e] - 1 else 0)
    cc = jnp.asarray(np.asarray(cc, np.int32))
    eid = jnp.asarray(np.asarray(eid, np.int32))
    first = jnp.asarray(np.asarray(first, np.int32))
    last = jnp.asarray(np.asarray(last, np.int32))

    grid_spec = pltpu.PrefetchScalarGridSpec(
        num_scalar_prefetch=4,
        grid=(NC,),
        in_specs=[
            pl.BlockSpec((T, pad), lambda s, cc, fr, la, ei: (0, cc[s])),
            pl.BlockSpec((pad, D), lambda s, cc, fr, la, ei: (cc[s], 0)),
            pl.BlockSpec((1, T, 1), lambda s, cc, fr, la, ei: (ei[s], 0, 0)),
        ],
        out_specs=pl.BlockSpec((T, D), lambda s, cc, fr, la, ei: (0, 0)),
        scratch_shapes=[pltpu.VMEM((T, D), jnp.float32)],
    )
    out = pl.pallas_call(
        functools.partial(_layer2_kernel, nd=max(1, D // 512)),
        grid_spec=grid_spec,
        out_shape=jax.ShapeDtypeStruct((T, D), jnp.float32),
        compiler_params=pltpu.CompilerParams(
            vmem_limit_bytes=63 * 1024 * 1024),
        interpret=interpret,
    )(cc, first, last, eid, a, W2p, gates_t.reshape(E, T, 1))
    return out + x


def kernel(x, router_w, router_b, *expert_params):
    w1_list = expert_params[0::4]
    w2_list = expert_params[2::4]
    return _moe_forward(x, router_w, router_b, list(w1_list), list(w2_list),
                        k_top=8, pad=256, token_block=512)


# speedup vs baseline: 1.0785x; 1.0029x over previous
import functools

import jax
import jax.numpy as jnp
import numpy as np
from jax import lax
from jax.experimental import pallas as pl
from jax.experimental.pallas import tpu as pltpu
from jax.experimental.pallas import tpu_sc as plsc

_EPS = 1e-5


def _logits_kernel(x_ref, rw_ref, rb_ref, lt_ref):
    logits = jnp.dot(x_ref[...], rw_ref[...],
                     preferred_element_type=jnp.float32)
    logits = logits + rb_ref[...]
    lt_ref[...] = logits.T


def _sc_routing_body(lt_hbm, gates_hbm, buf, obuf, *, E, TPW, k_top, nc):
    wid = lax.axis_index("s") * nc + lax.axis_index("c")
    base = wid * TPW
    pltpu.sync_copy(lt_hbm.at[:, pl.ds(base, TPW)], buf)

    @pl.loop(0, TPW // 16)
    def _group(g):
            sl = pl.ds(g * 16, 16)
            l = [buf[j, sl] for j in range(E)]
            orig = list(l)
            sel = [None] * E
            mx = None
            for k in range(k_top):
                m = l[0]
                arg = jnp.zeros((16,), jnp.float32)
                for j in range(1, E):
                    better = l[j] > m
                    m = jnp.where(better, l[j], m)
                    arg = jnp.where(better, float(j), arg)
                if k == 0:
                    mx = m
                for j in range(E):
                    pick = arg == float(j)
                    sel[j] = pick if sel[j] is None else (sel[j] | pick)
                    l[j] = jnp.where(pick, -1.0e30, l[j])
            ssum = jnp.zeros((16,), jnp.float32)
            es = []
            for j in range(E):
                e = jnp.where(sel[j], jnp.exp(orig[j] - mx), 0.0)
                es.append(e)
                ssum = ssum + e
            inv = 1.0 / ssum
            for j in range(E):
                obuf[j, sl] = es[j] * inv
    pltpu.sync_copy(obuf, gates_hbm.at[:, pl.ds(base, TPW)])


def _sc_routing(logits_t, k_top):
    E, T = logits_t.shape
    info = plsc.get_sparse_core_info()
    nc, ns = info.num_cores, info.num_subcores
    nw = nc * ns
    TPW = T // nw
    mesh = plsc.VectorSubcoreMesh(core_axis_name="c", subcore_axis_name="s")
    fn = functools.partial(
        pl.kernel,
        mesh=mesh,
        out_type=jax.ShapeDtypeStruct((E, T), jnp.float32),
        scratch_types=[
            pltpu.VMEM((E, TPW), jnp.float32),
            pltpu.VMEM((E, TPW), jnp.float32),
        ],
    )(functools.partial(_sc_routing_body, E=E, TPW=TPW, k_top=k_top, nc=nc))
    return fn(logits_t)


def _col_stats(h):
    n = h.shape[0]
    m = jnp.mean(h, axis=0, keepdims=True)
    v = jnp.mean(jnp.square(h - m), axis=0, keepdims=True)
    return m, v


def _layer1_kernel(x_ref, w1_ref, a_ref):
    h = jnp.dot(x_ref[...], w1_ref[...], preferred_element_type=jnp.float32)
    m, v = _col_stats(h)
    a = jnp.maximum((h - m) * jax.lax.rsqrt(v + _EPS), 0.0)
    a_ref[...] = a.astype(a_ref.dtype)


def _layer2_kernel(cc_ref, first_ref, last_ref, eid_ref,
                   a_ref, w2_ref, gates_ref, out_ref, h2_ref, *, nd):
    s = pl.program_id(0)
    D = w2_ref.shape[1]
    dw = D // nd

    @pl.when(first_ref[s] == 1)
    def _():
        h2_ref[...] = jnp.zeros_like(h2_ref)

    @pl.when(s == 0)
    def _():
        out_ref[...] = jnp.zeros_like(out_ref)

    a_blk = a_ref[...]
    for dh in range(nd):
        h2_ref[:, dh * dw:(dh + 1) * dw] += jnp.dot(
            a_blk, w2_ref[:, dh * dw:(dh + 1) * dw],
            preferred_element_type=jnp.float32)

    @pl.when(last_ref[s] == 1)
    def _():
        g = gates_ref[0]
        for dh in range(nd):
            h2 = h2_ref[:, dh * dw:(dh + 1) * dw]
            m, v = _col_stats(h2)
            bn = (h2 - m) * jax.lax.rsqrt(v + _EPS)
            out_ref[:, dh * dw:(dh + 1) * dw] += g * bn


def _moe_forward(x, router_w, router_b, w1_list, w2_list, *,
                 k_top, pad, token_block, interpret=False):
    T, D = x.shape
    E = len(w1_list)
    sizes = [int(w.shape[1]) for w in w1_list]
    psizes = [-(-s // pad) * pad for s in sizes]
    S = int(sum(psizes))
    nchunks = [ps // pad for ps in psizes]
    NC = int(sum(nchunks))

    logits_t = pl.pallas_call(
        _logits_kernel,
        grid=(T // token_block,),
        in_specs=[
            pl.BlockSpec((token_block, D), lambda i: (i, 0)),
            pl.BlockSpec((D, E), lambda i: (0, 0)),
            pl.BlockSpec((1, E), lambda i: (0, 0)),
        ],
        out_specs=pl.BlockSpec((E, token_block), lambda i: (0, i)),
        out_shape=jax.ShapeDtypeStruct((E, T), jnp.float32),
        interpret=interpret,
    )(x, router_w, router_b.reshape(1, E))

    W1p = jnp.concatenate(
        [jnp.pad(w, ((0, 0), (0, ps - s)))
         for w, s, ps in zip(w1_list, sizes, psizes)],
        axis=1).astype(jnp.bfloat16)
    W2p = jnp.concatenate(
        [jnp.pad(w, ((0, ps - s), (0, 0)))
         for w, s, ps in zip(w2_list, sizes, psizes)],
        axis=0).astype(jnp.bfloat16)
    x_bf = x.astype(jnp.bfloat16)

    a = pl.pallas_call(
        _layer1_kernel,
        grid=(NC,),
        in_specs=[
            pl.BlockSpec((T, D), lambda j: (0, 0)),
            pl.BlockSpec((D, pad), lambda j: (0, j)),
        ],
        out_specs=pl.BlockSpec((T, pad), lambda j: (0, j)),
        out_shape=jax.ShapeDtypeStruct((T, S), jnp.bfloat16),
        interpret=interpret,
    )(x_bf, W1p)

    gates_t = _sc_routing(logits_t, k_top)

    cc, eid, first, last = [], [], [], []
    for e in range(E):
        base = sum(nchunks[:e])
        for j in range(nchunks[e]):
            cc.append(base + j)
            eid.append(e)
            first.append(1 if j == 0 else 0)
            last.append(1 if j == nchunks[e] - 1 else 0)
    cc = jnp.asarray(np.asarray(cc, np.int32))
    eid = jnp.asarray(np.asarray(eid, np.int32))
    first = jnp.asarray(np.asarray(first, np.int32))
    last = jnp.asarray(np.asarray(last, np.int32))

    grid_spec = pltpu.PrefetchScalarGridSpec(
        num_scalar_prefetch=4,
        grid=(NC,),
        in_specs=[
            pl.BlockSpec((T, pad), lambda s, cc, fr, la, ei: (0, cc[s])),
            pl.BlockSpec((pad, D), lambda s, cc, fr, la, ei: (cc[s], 0)),
            pl.BlockSpec((1, T, 1), lambda s, cc, fr, la, ei: (ei[s], 0, 0)),
        ],
        out_specs=pl.BlockSpec((T, D), lambda s, cc, fr, la, ei: (0, 0)),
        scratch_shapes=[pltpu.VMEM((T, D), jnp.float32)],
    )
    out = pl.pallas_call(
        functools.partial(_layer2_kernel, nd=max(1, D // 512)),
        grid_spec=grid_spec,
        out_shape=jax.ShapeDtypeStruct((T, D), jnp.float32),
        compiler_params=pltpu.CompilerParams(
            vmem_limit_bytes=63 * 1024 * 1024),
        interpret=interpret,
    )(cc, first, last, eid, a, W2p, gates_t.reshape(E, T, 1))
    return out + x


def kernel(x, router_w, router_b, *expert_params):
    w1_list = expert_params[0::4]
    w2_list = expert_params[2::4]
    return _moe_forward(x, router_w, router_b, list(w1_list), list(w2_list),
                        k_top=8, pad=256, token_block=512)


# speedup vs baseline: 1.0806x; 1.0019x over previous
import functools

import jax
import jax.numpy as jnp
import numpy as np
from jax import lax
from jax.experimental import pallas as pl
from jax.experimental.pallas import tpu as pltpu
from jax.experimental.pallas import tpu_sc as plsc

_EPS = 1e-5


def _logits_kernel(x_ref, rw_ref, rb_ref, lt_ref):
    logits = jnp.dot(x_ref[...], rw_ref[...],
                     preferred_element_type=jnp.float32)
    logits = logits + rb_ref[...]
    lt_ref[...] = logits.T


def _sc_routing_body(lt_hbm, gates_hbm, buf, obuf, *, E, TPW, k_top, nc):
    wid = lax.axis_index("s") * nc + lax.axis_index("c")
    base = wid * TPW
    pltpu.sync_copy(lt_hbm.at[:, pl.ds(base, TPW)], buf)

    @pl.loop(0, TPW // 16)
    def _group(g):
            sl = pl.ds(g * 16, 16)
            l = [buf[j, sl] for j in range(E)]
            orig = list(l)
            sel = [None] * E
            mx = None
            for k in range(k_top):
                m = l[0]
                arg = jnp.zeros((16,), jnp.float32)
                for j in range(1, E):
                    better = l[j] > m
                    m = jnp.where(better, l[j], m)
                    arg = jnp.where(better, float(j), arg)
                if k == 0:
                    mx = m
                for j in range(E):
                    pick = arg == float(j)
                    sel[j] = pick if sel[j] is None else (sel[j] | pick)
                    l[j] = jnp.where(pick, -1.0e30, l[j])
            ssum = jnp.zeros((16,), jnp.float32)
            es = []
            for j in range(E):
                e = jnp.where(sel[j], jnp.exp(orig[j] - mx), 0.0)
                es.append(e)
                ssum = ssum + e
            inv = 1.0 / ssum
            for j in range(E):
                obuf[j, sl] = es[j] * inv
    pltpu.sync_copy(obuf, gates_hbm.at[:, pl.ds(base, TPW)])


def _sc_routing(logits_t, k_top):
    E, T = logits_t.shape
    info = plsc.get_sparse_core_info()
    nc, ns = info.num_cores, info.num_subcores
    nw = nc * ns
    TPW = T // nw
    mesh = plsc.VectorSubcoreMesh(core_axis_name="c", subcore_axis_name="s")
    fn = functools.partial(
        pl.kernel,
        mesh=mesh,
        out_type=jax.ShapeDtypeStruct((E, T), jnp.float32),
        scratch_types=[
            pltpu.VMEM((E, TPW), jnp.float32),
            pltpu.VMEM((E, TPW), jnp.float32),
        ],
    )(functools.partial(_sc_routing_body, E=E, TPW=TPW, k_top=k_top, nc=nc))
    return fn(logits_t)


def _col_stats(h):
    n = h.shape[0]
    m = jnp.mean(h, axis=0, keepdims=True)
    v = jnp.mean(jnp.square(h - m), axis=0, keepdims=True)
    return m, v


def _layer1_kernel(x_ref, w1_ref, a_ref):
    h = jnp.dot(x_ref[...], w1_ref[...], preferred_element_type=jnp.float32)
    m, v = _col_stats(h)
    a = jnp.maximum((h - m) * jax.lax.rsqrt(v + _EPS), 0.0)
    a_ref[...] = a.astype(a_ref.dtype)


def _layer2_kernel(cc_ref, first_ref, last_ref, eid_ref,
                   a_ref, w2_ref, gates_ref, out_ref, h2_ref, *, nd):
    s = pl.program_id(0)
    D = w2_ref.shape[1]
    dw = D // nd

    @pl.when(first_ref[s] == 1)
    def _():
        h2_ref[...] = jnp.zeros_like(h2_ref)

    @pl.when(s == 0)
    def _():
        out_ref[...] = jnp.zeros_like(out_ref)

    a_blk = a_ref[...]
    for dh in range(nd):
        h2_ref[:, dh * dw:(dh + 1) * dw] += jnp.dot(
            a_blk, w2_ref[:, dh * dw:(dh + 1) * dw],
            preferred_element_type=jnp.float32)

    @pl.when(last_ref[s] == 1)
    def _():
        g = gates_ref[0]
        for dh in range(nd):
            h2 = h2_ref[:, dh * dw:(dh + 1) * dw]
            m, v = _col_stats(h2)
            bn = (h2 - m) * jax.lax.rsqrt(v + _EPS)
            out_ref[:, dh * dw:(dh + 1) * dw] += g * bn


def _moe_forward(x, router_w, router_b, w1_list, w2_list, *, k_top, pad, token_block):
    T, D = x.shape
    E = len(w1_list)
    sizes = [int(w.shape[1]) for w in w1_list]
    psizes = [-(-s // pad) * pad for s in sizes]
    S = int(sum(psizes))
    nchunks = [ps // pad for ps in psizes]
    NC = int(sum(nchunks))

    logits_t = pl.pallas_call(
        _logits_kernel,
        grid=(T // token_block,),
        in_specs=[
            pl.BlockSpec((token_block, D), lambda i: (i, 0)),
            pl.BlockSpec((D, E), lambda i: (0, 0)),
            pl.BlockSpec((1, E), lambda i: (0, 0)),
        ],
        out_specs=pl.BlockSpec((E, token_block), lambda i: (0, i)),
        out_shape=jax.ShapeDtypeStruct((E, T), jnp.float32),
    )(x, router_w, router_b.reshape(1, E))

    W1p = jnp.concatenate(
        [jnp.pad(w, ((0, 0), (0, ps - s)))
         for w, s, ps in zip(w1_list, sizes, psizes)],
        axis=1).astype(jnp.bfloat16)
    W2p = jnp.concatenate(
        [jnp.pad(w, ((0, ps - s), (0, 0)))
         for w, s, ps in zip(w2_list, sizes, psizes)],
        axis=0).astype(jnp.bfloat16)
    x_bf = x.astype(jnp.bfloat16)

    a = pl.pallas_call(
        _layer1_kernel,
        grid=(NC,),
        in_specs=[
            pl.BlockSpec((T, D), lambda j: (0, 0)),
            pl.BlockSpec((D, pad), lambda j: (0, j)),
        ],
        out_specs=pl.BlockSpec((T, pad), lambda j: (0, j)),
        out_shape=jax.ShapeDtypeStruct((T, S), jnp.bfloat16),
    )(x_bf, W1p)

    gates_t = _sc_routing(logits_t, k_top)

    cc, eid, first, last = [], [], [], []
    for e in range(E):
        base = sum(nchunks[:e])
        for j in range(nchunks[e]):
            cc.append(base + j)
            eid.append(e)
            first.append(1 if j == 0 else 0)
            last.append(1 if j == nchunks[e] - 1 else 0)
    cc = jnp.asarray(np.asarray(cc, np.int32))
    eid = jnp.asarray(np.asarray(eid, np.int32))
    first = jnp.asarray(np.asarray(first, np.int32))
    last = jnp.asarray(np.asarray(last, np.int32))

    grid_spec = pltpu.PrefetchScalarGridSpec(
        num_scalar_prefetch=4,
        grid=(NC,),
        in_specs=[
            pl.BlockSpec((T, pad), lambda s, cc, fr, la, ei: (0, cc[s])),
            pl.BlockSpec((pad, D), lambda s, cc, fr, la, ei: (cc[s], 0)),
            pl.BlockSpec((1, T, 1), lambda s, cc, fr, la, ei: (ei[s], 0, 0)),
        ],
        out_specs=pl.BlockSpec((T, D), lambda s, cc, fr, la, ei: (0, 0)),
        scratch_shapes=[pltpu.VMEM((T, D), jnp.float32)],
    )
    out = pl.pallas_call(
        functools.partial(_layer2_kernel, nd=max(1, D // 512)),
        grid_spec=grid_spec,
        out_shape=jax.ShapeDtypeStruct((T, D), jnp.float32),
        compiler_params=pltpu.CompilerParams(
            vmem_limit_bytes=63 * 1024 * 1024),
    )(cc, first, last, eid, a, W2p, gates_t.reshape(E, T, 1))
    return out + x


def kernel(x, router_w, router_b, *expert_params):
    w1_list = expert_params[0::4]
    w2_list = expert_params[2::4]
    return _moe_forward(x, router_w, router_b, list(w1_list), list(w2_list),
                        k_top=8, pad=256, token_block=512)
